# Initial kernel scaffold; baseline (speedup 1.0000x reference)
import jax
import jax.numpy as jnp
from jax.experimental import pallas as pl


def kernel(r, edge_index, phi_density, phi_pair, emb_weights):
    raise NotImplementedError("write your pallas kernel here")


# speedup vs baseline: 9.0341x; 9.0341x over previous
import functools

import jax
import jax.numpy as jnp
from jax import lax
from jax.experimental import pallas as pl
from jax.experimental.pallas import tpu as pltpu
from jax.experimental.pallas import tpu_sc as plsc

NBASIS = 128
CUTOFF = 6.0
N_NODES = 10000
N_EDGES = 320000
NPAD = 10240
PI = 3.141592653589793

DELTA = CUTOFF / (NBASIS - 1)
GAMMA = 1.0 / DELTA

NC = 2
NS = 16
LANES = 16
NW = NC * NS
E_PER_W = N_EDGES // NW
SUBCH = 2000
NSUB = E_PER_W // SUBCH
SLICE = NPAD // NS

SF0, SF1, SF2, SF3, SF4 = 2.0, 1.0, 0.5, 1.0 / 6.0, 1.0 / 24.0

BE = 512


def _edge_body(r_ref, pd_ref, pp_ref, a_ref, p_ref, q_ref, pair_ref):
    rx = r_ref[0:1, :]
    ry = r_ref[1:2, :]
    rz = r_ref[2:3, :]
    l2 = rx * rx + ry * ry + rz * rz
    li = lax.rsqrt(l2)
    l = l2 * li
    c = lax.broadcasted_iota(jnp.int32, (NBASIS, 1), 0).astype(jnp.float32) * DELTA
    pd = pd_ref[:, :]
    s = jnp.maximum(pd, 0.0) + jnp.log(1.0 + jnp.exp(-jnp.abs(pd)))
    p = pp_ref[:, :]
    u = jnp.exp((-GAMMA) * (l - c) ** 2)
    ang = (PI / CUTOFF) * l
    fc = 0.5 * (1.0 + jnp.cos(ang))
    fcp = (-PI / (2.0 * CUTOFF)) * jnp.sin(ang)
    m0 = jnp.sum(u * s, axis=0, keepdims=True)
    m1 = jnp.sum(u * p, axis=0, keepdims=True)
    m2 = jnp.sum(u * (s * c), axis=0, keepdims=True)
    m3 = jnp.sum(u * (p * c), axis=0, keepdims=True)
    halffc = 0.5 * fc
    dens = halffc * m0
    cpair = halffc * m1
    t = 0.5 * fcp - GAMMA * l * fc
    bb = t * m0 + GAMMA * fc * m2
    dd = t * m1 + GAMMA * fc * m3
    a_ref[...] = dens[0]
    p_ref[...] = (bb * li)[0]
    li2 = li * li
    q_ref[...] = (dd * li2 - cpair * li * li2)[0]

    @pl.when(pl.program_id(0) == 0)
    def _():
        pair_ref[...] = jnp.zeros((1, 1), jnp.float32)

    pair_ref[...] += jnp.sum(cpair * li).reshape(1, 1)


def _edge_stage(r_t, phi_d, phi_p):
    grid = N_EDGES // BE
    return pl.pallas_call(
        _edge_body,
        grid=(grid,),
        in_specs=[
            pl.BlockSpec((3, BE), lambda i: (0, i)),
            pl.BlockSpec((NBASIS, 1), lambda i: (0, 0)),
            pl.BlockSpec((NBASIS, 1), lambda i: (0, 0)),
        ],
        out_specs=[
            pl.BlockSpec((BE,), lambda i: (i,)),
            pl.BlockSpec((BE,), lambda i: (i,)),
            pl.BlockSpec((BE,), lambda i: (i,)),
            pl.BlockSpec((1, 1), lambda i: (0, 0)),
        ],
        out_shape=[
            jax.ShapeDtypeStruct((N_EDGES,), jnp.float32),
            jax.ShapeDtypeStruct((N_EDGES,), jnp.float32),
            jax.ShapeDtypeStruct((N_EDGES,), jnp.float32),
            jax.ShapeDtypeStruct((1, 1), jnp.float32),
        ],
    )(r_t, phi_d, phi_p)


def _node_body(rho_ref, ew_ref, fp_ref, fsum_ref):
    rho = rho_ref[0] + rho_ref[1]
    w0 = ew_ref[0, 0]
    w1 = ew_ref[0, 1]
    w2 = ew_ref[0, 2]
    w3 = ew_ref[0, 3]
    w4 = ew_ref[0, 4]
    sq = jnp.sqrt(rho)
    rho2 = rho * rho
    f = ((SF0 * w0) * sq + (SF1 * w1) * rho + (SF2 * w2) * rho2
         + (SF3 * w3) * rho2 * rho + (SF4 * w4) * rho2 * rho2)
    fsum_ref[...] = jnp.sum(f).reshape(1, 1)
    fp_ref[...] = (w0 * lax.rsqrt(rho) + SF1 * w1 + (2.0 * SF2 * w2) * rho
                   + (3.0 * SF3 * w3) * rho2 + (4.0 * SF4 * w4) * rho2 * rho)


def _node_stage(rho_parts, ew):
    nrows = NPAD // 128
    return pl.pallas_call(
        _node_body,
        out_shape=[
            jax.ShapeDtypeStruct((nrows, 128), jnp.float32),
            jax.ShapeDtypeStruct((1, 1), jnp.float32),
        ],
    )(rho_parts.reshape(2, nrows, 128), ew)


def _seg_body(dst_hbm, a_hbm, out_hbm, idx_v, val_v, acc_v, tmp_v, red_v, sh_v):
    cid = lax.axis_index("c")
    sid = lax.axis_index("s")
    wid = sid * NC + cid
    base = wid * E_PER_W

    pltpu.sync_copy(dst_hbm.at[pl.ds(base, E_PER_W)], idx_v)
    pltpu.sync_copy(a_hbm.at[pl.ds(base, E_PER_W)], val_v)

    zeros = jnp.zeros((LANES,), jnp.float32)

    def _zero(i, _):
        acc_v[pl.ds(i * LANES, LANES)] = zeros
        return 0

    lax.fori_loop(0, NPAD // LANES, _zero, 0)

    def _scat(i, _):
        sl = pl.ds(i * LANES, LANES)
        plsc.addupdate_scatter(acc_v, [idx_v[sl]], val_v[sl])
        return 0

    lax.fori_loop(0, E_PER_W // LANES, _scat, 0)

    pltpu.sync_copy(acc_v, sh_v.at[sid])
    plsc.subcore_barrier()

    off = sid * SLICE
    zeros_s = jnp.zeros((LANES,), jnp.float32)

    def _zr(i, _):
        red_v[pl.ds(i * LANES, LANES)] = zeros_s
        return 0

    lax.fori_loop(0, SLICE // LANES, _zr, 0)

    for t in range(NS):
        pltpu.sync_copy(sh_v.at[t, pl.ds(off, SLICE)], tmp_v)

        def _acc(i, _):
            sl = pl.ds(i * LANES, LANES)
            red_v[sl] += tmp_v[sl]
            return 0

        lax.fori_loop(0, SLICE // LANES, _acc, 0)

    pltpu.sync_copy(red_v, out_hbm.at[cid, pl.ds(off, SLICE)])


def _seg_sum(dst, a):
    mesh = plsc.VectorSubcoreMesh(
        core_axis_name="c", subcore_axis_name="s", num_cores=NC, num_subcores=NS
    )
    fn = pl.kernel(
        _seg_body,
        out_type=jax.ShapeDtypeStruct((2, NPAD), jnp.float32),
        mesh=mesh,
        compiler_params=pltpu.CompilerParams(needs_layout_passes=False),
        scratch_types=[
            pltpu.VMEM((E_PER_W,), jnp.int32),
            pltpu.VMEM((E_PER_W,), jnp.float32),
            pltpu.VMEM((NPAD,), jnp.float32),
            pltpu.VMEM((SLICE,), jnp.float32),
            pltpu.VMEM((SLICE,), jnp.float32),
            pltpu.VMEM_SHARED((NS, NPAD), jnp.float32),
        ],
    )
    return fn(dst, a)


def _force_body(fp_hbm, dst_hbm, src_hbm, p_hbm, q_hbm, rx_hbm, ry_hbm, rz_hbm,
                out_hbm, fp_tab, di_v, si_v, p_v, q_v, rx_v, ry_v, rz_v,
                fx_v, fy_v, fz_v, tmp_v, red_v, sh_v):
    cid = lax.axis_index("c")
    sid = lax.axis_index("s")
    wid = sid * NC + cid
    base = wid * E_PER_W

    pltpu.sync_copy(fp_hbm, fp_tab)

    zeros = jnp.zeros((LANES,), jnp.float32)

    def _zero(i, _):
        sl = pl.ds(i * LANES, LANES)
        fx_v[sl] = zeros
        fy_v[sl] = zeros
        fz_v[sl] = zeros
        return 0

    lax.fori_loop(0, NPAD // LANES, _zero, 0)

    for k in range(NSUB):
        cb = base + k * SUBCH
        pltpu.sync_copy(dst_hbm.at[pl.ds(cb, SUBCH)], di_v)
        pltpu.sync_copy(src_hbm.at[pl.ds(cb, SUBCH)], si_v)
        pltpu.sync_copy(p_hbm.at[pl.ds(cb, SUBCH)], p_v)
        pltpu.sync_copy(q_hbm.at[pl.ds(cb, SUBCH)], q_v)
        pltpu.sync_copy(rx_hbm.at[pl.ds(cb, SUBCH)], rx_v)
        pltpu.sync_copy(ry_hbm.at[pl.ds(cb, SUBCH)], ry_v)
        pltpu.sync_copy(rz_hbm.at[pl.ds(cb, SUBCH)], rz_v)

        def _edge(i, _):
            sl = pl.ds(i * LANES, LANES)
            didx = di_v[sl]
            sidx = si_v[sl]
            fpe = plsc.load_gather(fp_tab, [didx])
            coef = fpe * p_v[sl] + q_v[sl]
            vx = coef * rx_v[sl]
            vy = coef * ry_v[sl]
            vz = coef * rz_v[sl]
            plsc.addupdate_scatter(fx_v, [didx], -vx)
            plsc.addupdate_scatter(fy_v, [didx], -vy)
            plsc.addupdate_scatter(fz_v, [didx], -vz)
            plsc.addupdate_scatter(fx_v, [sidx], vx)
            plsc.addupdate_scatter(fy_v, [sidx], vy)
            plsc.addupdate_scatter(fz_v, [sidx], vz)
            return 0

        lax.fori_loop(0, SUBCH // LANES, _edge, 0)

    pltpu.sync_copy(fx_v, sh_v.at[sid, pl.ds(0, NPAD)])
    pltpu.sync_copy(fy_v, sh_v.at[sid, pl.ds(NPAD, NPAD)])
    pltpu.sync_copy(fz_v, sh_v.at[sid, pl.ds(2 * NPAD, NPAD)])
    plsc.subcore_barrier()

    off = sid * SLICE
    for c3 in range(3):

        def _zr(i, _):
            red_v[pl.ds(i * LANES, LANES)] = zeros
            return 0

        lax.fori_loop(0, SLICE // LANES, _zr, 0)

        for t in range(NS):
            pltpu.sync_copy(sh_v.at[t, pl.ds(c3 * NPAD + off, SLICE)], tmp_v)

            def _acc(i, _):
                sl = pl.ds(i * LANES, LANES)
                red_v[sl] += tmp_v[sl]
                return 0

            lax.fori_loop(0, SLICE // LANES, _acc, 0)

        pltpu.sync_copy(red_v, out_hbm.at[cid, pl.ds(c3 * NPAD + off, SLICE)])


def _force_stage(fp, dst, src, pcoef, qcoef, rx, ry, rz):
    mesh = plsc.VectorSubcoreMesh(
        core_axis_name="c", subcore_axis_name="s", num_cores=NC, num_subcores=NS
    )
    fn = pl.kernel(
        _force_body,
        out_type=jax.ShapeDtypeStruct((2, 3 * NPAD), jnp.float32),
        mesh=mesh,
        compiler_params=pltpu.CompilerParams(needs_layout_passes=False),
        scratch_types=[
            pltpu.VMEM((NPAD,), jnp.float32),
            pltpu.VMEM((SUBCH,), jnp.int32),
            pltpu.VMEM((SUBCH,), jnp.int32),
            pltpu.VMEM((SUBCH,), jnp.float32),
            pltpu.VMEM((SUBCH,), jnp.float32),
            pltpu.VMEM((SUBCH,), jnp.float32),
            pltpu.VMEM((SUBCH,), jnp.float32),
            pltpu.VMEM((SUBCH,), jnp.float32),
            pltpu.VMEM((NPAD,), jnp.float32),
            pltpu.VMEM((NPAD,), jnp.float32),
            pltpu.VMEM((NPAD,), jnp.float32),
            pltpu.VMEM((SLICE,), jnp.float32),
            pltpu.VMEM((SLICE,), jnp.float32),
            pltpu.VMEM_SHARED((NS, 3 * NPAD), jnp.float32),
        ],
    )
    return fn(fp, dst, src, pcoef, qcoef, rx, ry, rz)


def kernel(r, edge_index, phi_density, phi_pair, emb_weights):
    r_t = r.T
    src = edge_index[0]
    dst = edge_index[1]
    phi_d = phi_density.reshape(NBASIS, 1)
    phi_p = phi_pair.reshape(NBASIS, 1)
    ew = jnp.concatenate([emb_weights, jnp.zeros((3,), jnp.float32)]).reshape(1, 8)

    a_e, p_e, q_e, pair_sum = _edge_stage(r_t, phi_d, phi_p)
    rho_parts = _seg_sum(dst, a_e)
    fp_2d, f_sum = _node_stage(rho_parts, ew)
    fp = fp_2d.reshape(NPAD)
    fparts = _force_stage(fp, dst, src, p_e, q_e, r_t[0], r_t[1], r_t[2])

    forces = (fparts[0] + fparts[1]).reshape(3, NPAD)[:, :N_NODES].T
    total_energy = (pair_sum[0, 0] + f_sum[0, 0]).reshape(1)
    return (total_energy, forces)
